# baseline (speedup 1.0000x reference)
import functools

import jax
import jax.numpy as jnp
from jax import lax
from jax.experimental import pallas as pl
from jax.experimental.pallas import tpu as pltpu
from jax.experimental.pallas import tpu_sc as plsc

VOCAB = 100000
EMB = 64
BATCH = 1024

_NUM_CORES = 2
_NUM_SUBCORES = 16
_NUM_WORKERS = _NUM_CORES * _NUM_SUBCORES
_B_PER_W = BATCH // _NUM_WORKERS

_VTILE = 2048


def _sc_gather(target, emb_table):
    mesh = plsc.VectorSubcoreMesh(core_axis_name="c", subcore_axis_name="s")

    @functools.partial(
        pl.kernel,
        mesh=mesh,
        compiler_params=pltpu.CompilerParams(use_tc_tiling_on_sc=False),
        out_type=jax.ShapeDtypeStruct((BATCH, EMB), jnp.float32),
        scratch_types=[
            pltpu.VMEM((_B_PER_W,), jnp.int32),
            pltpu.VMEM((_B_PER_W, EMB), jnp.float32),
            pltpu.SemaphoreType.DMA,
        ],
    )
    def gather_kernel(idx_hbm, table_hbm, out_hbm, idx_v, rows_v, sem):
        wid = lax.axis_index("s") * _NUM_CORES + lax.axis_index("c")
        base = wid * _B_PER_W
        pltpu.sync_copy(idx_hbm.at[pl.ds(base, _B_PER_W)], idx_v)
        pltpu.async_copy(table_hbm.at[idx_v], rows_v, sem).wait()
        pltpu.sync_copy(rows_v, out_hbm.at[pl.ds(base, _B_PER_W)])

    return gather_kernel(target, emb_table)


def _proj_body(embed_ref, w_ref, b_ref, out_ref):
    acc = lax.dot_general(
        embed_ref[...],
        w_ref[...],
        (((1,), (1,)), ((), ())),
        preferred_element_type=jnp.float32,
    )
    out_ref[...] = acc + b_ref[...]


def _tc_project(embed, W, b2d):
    n_tiles = pl.cdiv(VOCAB, _VTILE)
    return pl.pallas_call(
        _proj_body,
        grid=(n_tiles,),
        in_specs=[
            pl.BlockSpec((BATCH, EMB), lambda v: (0, 0)),
            pl.BlockSpec((_VTILE, EMB), lambda v: (v, 0)),
            pl.BlockSpec((1, _VTILE), lambda v: (0, v)),
        ],
        out_specs=pl.BlockSpec((BATCH, _VTILE), lambda v: (0, v)),
        out_shape=jax.ShapeDtypeStruct((BATCH, VOCAB), jnp.float32),
    )(embed, W, b2d)


def kernel(target, emb_table, W, b):
    embed = _sc_gather(target.astype(jnp.int32), emb_table)
    return _tc_project(embed, W, b.reshape(1, VOCAB))


# speedup vs baseline: 1.0002x; 1.0002x over previous
import functools

import jax
import jax.numpy as jnp
from jax import lax
from jax.experimental import pallas as pl
from jax.experimental.pallas import tpu as pltpu
from jax.experimental.pallas import tpu_sc as plsc

VOCAB = 100000
EMB = 64
BATCH = 1024
PAIR = 2 * EMB

_NUM_CORES = 2
_NUM_SUBCORES = 16
_NUM_WORKERS = _NUM_CORES * _NUM_SUBCORES
_B_PER_W = BATCH // _NUM_WORKERS

_VTILE = 2048


def _sc_gather_pairs(idx2, table2):
    mesh = plsc.VectorSubcoreMesh(core_axis_name="c", subcore_axis_name="s")

    @functools.partial(
        pl.kernel,
        mesh=mesh,
        out_type=jax.ShapeDtypeStruct((BATCH, PAIR), jnp.float32),
        scratch_types=[
            pltpu.VMEM((_B_PER_W,), jnp.int32),
            pltpu.VMEM((_B_PER_W, PAIR), jnp.float32),
            pltpu.SemaphoreType.DMA,
        ],
    )
    def gather_kernel(idx_hbm, table_hbm, out_hbm, idx_v, rows_v, sem):
        wid = lax.axis_index("s") * _NUM_CORES + lax.axis_index("c")
        base = wid * _B_PER_W
        pltpu.sync_copy(idx_hbm.at[pl.ds(base, _B_PER_W)], idx_v)
        pltpu.async_copy(table_hbm.at[idx_v], rows_v, sem).wait()
        pltpu.sync_copy(rows_v, out_hbm.at[pl.ds(base, _B_PER_W)])

    return gather_kernel(idx2, table2)


def _proj_body(par_ref, pairs_ref, w_ref, b_ref, out_ref, emb_ref):
    @pl.when(pl.program_id(0) == 0)
    def _select_half():
        left = pairs_ref[:, :EMB]
        right = pairs_ref[:, EMB:]
        emb_ref[...] = jnp.where(par_ref[...] == 1, right, left)

    acc = lax.dot_general(
        emb_ref[...],
        w_ref[...],
        (((1,), (1,)), ((), ())),
        preferred_element_type=jnp.float32,
    )
    out_ref[...] = acc + b_ref[...]


def _tc_project(parity, pairs, W, b2d):
    n_vtiles = pl.cdiv(VOCAB, _VTILE)
    return pl.pallas_call(
        _proj_body,
        grid=(n_vtiles,),
        in_specs=[
            pl.BlockSpec((BATCH, 1), lambda v: (0, 0)),
            pl.BlockSpec((BATCH, PAIR), lambda v: (0, 0)),
            pl.BlockSpec((_VTILE, EMB), lambda v: (v, 0)),
            pl.BlockSpec((1, _VTILE), lambda v: (0, v)),
        ],
        out_specs=pl.BlockSpec((BATCH, _VTILE), lambda v: (0, v)),
        out_shape=jax.ShapeDtypeStruct((BATCH, VOCAB), jnp.float32),
        scratch_shapes=[pltpu.VMEM((BATCH, EMB), jnp.float32)],
    )(parity, pairs, W, b2d)


def kernel(target, emb_table, W, b):
    target = target.astype(jnp.int32)
    table2 = emb_table.reshape(VOCAB // 2, PAIR)
    pairs = _sc_gather_pairs(target // 2, table2)
    parity = (target % 2).reshape(BATCH, 1)
    return _tc_project(parity, pairs, W, b.reshape(1, VOCAB))


# speedup vs baseline: 3.0541x; 3.0535x over previous
import functools

import jax
import jax.numpy as jnp
from jax import lax
from jax.experimental import pallas as pl
from jax.experimental.pallas import tpu as pltpu
from jax.experimental.pallas import tpu_sc as plsc

VOCAB = 100000
EMB = 64
BATCH = 1024

_NUM_CORES = 2
_NUM_SUBCORES = 16
_NUM_WORKERS = _NUM_CORES * _NUM_SUBCORES
_ROWS_PER_W = EMB // _NUM_WORKERS
_LANES = 16

_VTILE = 2048


def _sc_gather_cols(target, table_t):
    mesh = plsc.VectorSubcoreMesh(core_axis_name="c", subcore_axis_name="s")

    @functools.partial(
        pl.kernel,
        mesh=mesh,
        compiler_params=pltpu.CompilerParams(
            use_tc_tiling_on_sc=False, needs_layout_passes=False
        ),
        out_type=jax.ShapeDtypeStruct((EMB, BATCH), jnp.float32),
        scratch_types=[
            pltpu.VMEM((BATCH,), jnp.int32),
            pltpu.VMEM((VOCAB,), jnp.float32),
            pltpu.VMEM((BATCH,), jnp.float32),
            pltpu.SemaphoreType.DMA,
        ],
    )
    def gather_kernel(idx_hbm, table_hbm, out_hbm, idx_v, row_v, out_v, sem):
        wid = lax.axis_index("s") * _NUM_CORES + lax.axis_index("c")
        pltpu.sync_copy(idx_hbm, idx_v)
        for r in range(_ROWS_PER_W):
            e = wid * _ROWS_PER_W + r
            pltpu.sync_copy(table_hbm.at[e], row_v)
            for k in range(BATCH // _LANES):
                sl = pl.ds(k * _LANES, _LANES)
                out_v[sl] = plsc.load_gather(row_v, [idx_v[sl]])
            pltpu.sync_copy(out_v, out_hbm.at[e])

    return gather_kernel(target, table_t)


def _proj_body(x_ref, wt_ref, b_ref, out_ref):
    acc = lax.dot_general(
        wt_ref[...],
        x_ref[...],
        (((0,), (0,)), ((), ())),
        preferred_element_type=jnp.float32,
    )
    out_ref[...] = acc + b_ref[...].T


def _tc_project_t(x, Wt, b2d):
    n_vtiles = pl.cdiv(VOCAB, _VTILE)
    return pl.pallas_call(
        _proj_body,
        grid=(n_vtiles,),
        in_specs=[
            pl.BlockSpec((EMB, BATCH), lambda v: (0, 0)),
            pl.BlockSpec((EMB, _VTILE), lambda v: (0, v)),
            pl.BlockSpec((1, _VTILE), lambda v: (0, v)),
        ],
        out_specs=pl.BlockSpec((_VTILE, BATCH), lambda v: (v, 0)),
        out_shape=jax.ShapeDtypeStruct((VOCAB, BATCH), jnp.float32),
    )(x, Wt, b2d)


def kernel(target, emb_table, W, b):
    target = target.astype(jnp.int32)
    x = _sc_gather_cols(target, emb_table.T)
    out_t = _tc_project_t(x, W.T, b.reshape(1, VOCAB))
    return out_t.T


# speedup vs baseline: 3.6946x; 1.2097x over previous
import functools

import jax
import jax.numpy as jnp
from jax import lax
from jax.experimental import pallas as pl
from jax.experimental.pallas import tpu as pltpu
from jax.experimental.pallas import tpu_sc as plsc

VOCAB = 100000
EMB = 64
BATCH = 1024

_NUM_CORES = 2
_NUM_SUBCORES = 16
_NUM_WORKERS = _NUM_CORES * _NUM_SUBCORES
_ROWS_PER_W = EMB // _NUM_WORKERS
_LANES = 16

_VTILE = 2048


def _sc_gather_cols(target, table_t):
    mesh = plsc.VectorSubcoreMesh(core_axis_name="c", subcore_axis_name="s")

    @functools.partial(
        pl.kernel,
        mesh=mesh,
        compiler_params=pltpu.CompilerParams(needs_layout_passes=False),
        out_type=jax.ShapeDtypeStruct((EMB, BATCH), jnp.float32),
        scratch_types=[
            pltpu.VMEM((BATCH,), jnp.int32),
            pltpu.VMEM((VOCAB,), jnp.float32),
            pltpu.VMEM((BATCH,), jnp.float32),
            pltpu.SemaphoreType.DMA,
        ],
    )
    def gather_kernel(idx_hbm, table_hbm, out_hbm, idx_v, row_v, out_v, sem):
        wid = lax.axis_index("s") * _NUM_CORES + lax.axis_index("c")
        pltpu.sync_copy(idx_hbm, idx_v)
        for r in range(_ROWS_PER_W):
            e = wid * _ROWS_PER_W + r
            pltpu.sync_copy(table_hbm.at[e], row_v)
            for k in range(BATCH // _LANES):
                sl = pl.ds(k * _LANES, _LANES)
                out_v[sl] = plsc.load_gather(row_v, [idx_v[sl]])
            pltpu.sync_copy(out_v, out_hbm.at[e])

    return gather_kernel(target, table_t)


def _proj_body(x_ref, wt_ref, b_ref, out_ref):
    acc = lax.dot_general(
        wt_ref[...],
        x_ref[...],
        (((0,), (0,)), ((), ())),
        preferred_element_type=jnp.float32,
    )
    out_ref[...] = acc + b_ref[...].T


def _tc_project_t(x, Wt, b2d):
    n_vtiles = pl.cdiv(VOCAB, _VTILE)
    return pl.pallas_call(
        _proj_body,
        grid=(n_vtiles,),
        in_specs=[
            pl.BlockSpec((EMB, BATCH), lambda v: (0, 0)),
            pl.BlockSpec((EMB, _VTILE), lambda v: (0, v)),
            pl.BlockSpec((1, _VTILE), lambda v: (0, v)),
        ],
        out_specs=pl.BlockSpec((_VTILE, BATCH), lambda v: (v, 0)),
        out_shape=jax.ShapeDtypeStruct((VOCAB, BATCH), jnp.float32),
    )(x, Wt, b2d)


def kernel(target, emb_table, W, b):
    target = target.astype(jnp.int32)
    x = _sc_gather_cols(target, emb_table.T)
    out_t = _tc_project_t(x, W.T, b.reshape(1, VOCAB))
    return out_t.T


# speedup vs baseline: 3.7439x; 1.0133x over previous
import functools

import jax
import jax.numpy as jnp
from jax import lax
from jax.experimental import pallas as pl
from jax.experimental.pallas import tpu as pltpu
from jax.experimental.pallas import tpu_sc as plsc

VOCAB = 100000
EMB = 64
BATCH = 1024

_NUM_CORES = 2
_NUM_SUBCORES = 16
_NUM_WORKERS = _NUM_CORES * _NUM_SUBCORES
_ROWS_PER_W = EMB // _NUM_WORKERS
_LANES = 16

_VTILE = 4096


def _sc_gather_cols(target, table_t):
    mesh = plsc.VectorSubcoreMesh(core_axis_name="c", subcore_axis_name="s")

    @functools.partial(
        pl.kernel,
        mesh=mesh,
        compiler_params=pltpu.CompilerParams(needs_layout_passes=False),
        out_type=jax.ShapeDtypeStruct((EMB, BATCH), jnp.float32),
        scratch_types=[
            pltpu.VMEM((BATCH,), jnp.int32),
            pltpu.VMEM((VOCAB,), jnp.float32),
            pltpu.VMEM((BATCH,), jnp.float32),
            pltpu.SemaphoreType.DMA,
        ],
    )
    def gather_kernel(idx_hbm, table_hbm, out_hbm, idx_v, row_v, out_v, sem):
        wid = lax.axis_index("s") * _NUM_CORES + lax.axis_index("c")
        pltpu.sync_copy(idx_hbm, idx_v)
        for r in range(_ROWS_PER_W):
            e = wid * _ROWS_PER_W + r
            pltpu.sync_copy(table_hbm.at[e], row_v)
            for k in range(BATCH // _LANES):
                sl = pl.ds(k * _LANES, _LANES)
                out_v[sl] = plsc.load_gather(row_v, [idx_v[sl]])
            pltpu.sync_copy(out_v, out_hbm.at[e])

    return gather_kernel(target, table_t)


def _proj_body(x_ref, wt_ref, b_ref, out_ref):
    acc = lax.dot_general(
        wt_ref[...],
        x_ref[...],
        (((0,), (0,)), ((), ())),
        preferred_element_type=jnp.float32,
    )
    out_ref[...] = acc + b_ref[...].T


def _tc_project_t(x, Wt, b2d):
    n_vtiles = pl.cdiv(VOCAB, _VTILE)
    return pl.pallas_call(
        _proj_body,
        grid=(n_vtiles,),
        in_specs=[
            pl.BlockSpec((EMB, BATCH), lambda v: (0, 0)),
            pl.BlockSpec((EMB, _VTILE), lambda v: (0, v)),
            pl.BlockSpec((1, _VTILE), lambda v: (0, v)),
        ],
        out_specs=pl.BlockSpec((_VTILE, BATCH), lambda v: (v, 0)),
        out_shape=jax.ShapeDtypeStruct((VOCAB, BATCH), jnp.float32),
    )(x, Wt, b2d)


def kernel(target, emb_table, W, b):
    target = target.astype(jnp.int32)
    x = _sc_gather_cols(target, emb_table.T)
    out_t = _tc_project_t(x, W.T, b.reshape(1, VOCAB))
    return out_t.T
